# speedup vs baseline: 2.9732x; 2.9732x over previous
import functools

import jax
import jax.numpy as jnp
from jax import lax
from jax.experimental import pallas as pl
from jax.experimental.pallas import tpu as pltpu
from jax.experimental.pallas import tpu_sc as plsc

N_NODES = 100000
D_FEAT = 128
N_READOUT = 16384

NC = 2
NS = 16
L = 16

HALF = N_READOUT // NC
E_PER_W = N_READOUT // NS
CHUNK = 128
NCHUNK = E_PER_W // CHUNK
TRASH = HALF
ACC_ROWS = HALF + 8
OUT_PER_W = HALF // NS

_mesh = plsc.VectorSubcoreMesh(
    core_axis_name="c", subcore_axis_name="s", num_cores=NC, num_subcores=NS
)


@functools.partial(
    pl.kernel,
    out_type=jax.ShapeDtypeStruct((N_READOUT, D_FEAT), jnp.float32),
    mesh=_mesh,
    scratch_types=[
        pltpu.VMEM((E_PER_W,), jnp.int32),
        pltpu.VMEM((E_PER_W,), jnp.int32),
        pltpu.VMEM((CHUNK,), jnp.int32),
        pltpu.VMEM((CHUNK, D_FEAT), jnp.float32),
        pltpu.VMEM_SHARED((ACC_ROWS, D_FEAT), jnp.float32),
        pltpu.SemaphoreType.DMA,
    ],
)
def _readout_kernel(x_hbm, src_hbm, dst_hbm, out_hbm,
                    src_v, dst_v, dloc_v, rows_v, acc_sh, sem):
    c = lax.axis_index("c")
    s = lax.axis_index("s")
    ebase = s * E_PER_W

    pltpu.sync_copy(src_hbm.at[pl.ds(ebase, E_PER_W)], src_v)
    pltpu.sync_copy(dst_hbm.at[pl.ds(ebase, E_PER_W)], dst_v)

    zero = jnp.zeros((L,), jnp.float32)

    def _zrow(i, carry):
        for q in range(D_FEAT // L):
            rows_v[i, pl.ds(q * L, L)] = zero
        return carry

    lax.fori_loop(0, CHUNK, _zrow, 0)
    for r in range(OUT_PER_W // CHUNK):
        pltpu.sync_copy(rows_v, acc_sh.at[pl.ds(s * OUT_PER_W + r * CHUNK, CHUNK)])
    plsc.subcore_barrier()

    base_local = c * HALF
    for j in range(NCHUNK):
        pltpu.async_copy(
            x_hbm.at[src_v.at[pl.ds(j * CHUNK, CHUNK)]], rows_v, sem
        ).wait()
        for i in range(CHUNK // L):
            d = dst_v[pl.ds(j * CHUNK + i * L, L)]
            dl = d - base_local
            ok = (dl >= 0) & (dl < HALF)
            dloc_v[pl.ds(i * L, L)] = jnp.where(ok, dl, TRASH)
        pltpu.sync_copy(rows_v, acc_sh.at[dloc_v], add=True)

    plsc.subcore_barrier()

    out_base = c * HALF + s * OUT_PER_W
    pltpu.sync_copy(
        acc_sh.at[pl.ds(s * OUT_PER_W, OUT_PER_W)],
        out_hbm.at[pl.ds(out_base, OUT_PER_W)],
    )


def kernel(x, readout_src, readout_dst):
    return _readout_kernel(
        x, readout_src.astype(jnp.int32), readout_dst.astype(jnp.int32)
    )


# speedup vs baseline: 3.4324x; 1.1544x over previous
import functools

import jax
import jax.numpy as jnp
from jax import lax
from jax.experimental import pallas as pl
from jax.experimental.pallas import tpu as pltpu
from jax.experimental.pallas import tpu_sc as plsc

N_NODES = 100000
D_FEAT = 128
N_READOUT = 16384

NC = 2
NS = 16
L = 16

HALF = N_READOUT // NC
E_PER_W = N_READOUT // NS
CHUNK = 128
NCHUNK = E_PER_W // CHUNK
TRASH = HALF
ACC_ROWS = HALF + 8
OUT_PER_W = HALF // NS

_mesh = plsc.VectorSubcoreMesh(
    core_axis_name="c", subcore_axis_name="s", num_cores=NC, num_subcores=NS
)


@functools.partial(
    pl.kernel,
    out_type=jax.ShapeDtypeStruct((N_READOUT, D_FEAT), jnp.float32),
    mesh=_mesh,
    scratch_types=[
        pltpu.VMEM((E_PER_W,), jnp.int32),
        pltpu.VMEM((E_PER_W,), jnp.int32),
        pltpu.VMEM((CHUNK,), jnp.int32),
        pltpu.VMEM((3, CHUNK, D_FEAT), jnp.float32),
        pltpu.VMEM_SHARED((ACC_ROWS, D_FEAT), jnp.float32),
        pltpu.SemaphoreType.DMA,
        pltpu.SemaphoreType.DMA,
        pltpu.SemaphoreType.DMA,
    ],
)
def _readout_kernel(x_hbm, src_hbm, dst_hbm, out_hbm,
                    src_v, dst_v, dloc_v, rows_v,
                    acc_sh, sem_a, sem_b, sem_c):
    c = lax.axis_index("c")
    s = lax.axis_index("s")
    ebase = s * E_PER_W
    sems = (sem_a, sem_b, sem_c)

    pltpu.sync_copy(src_hbm.at[pl.ds(ebase, E_PER_W)], src_v)
    pltpu.sync_copy(dst_hbm.at[pl.ds(ebase, E_PER_W)], dst_v)

    def _gather(j):
        return pltpu.async_copy(
            x_hbm.at[src_v.at[pl.ds(j * CHUNK, CHUNK)]],
            rows_v.at[j % 3], sems[j % 3],
        )

    desc = [None] * NCHUNK
    desc[0] = _gather(0)
    desc[1] = _gather(1)

    zero = jnp.zeros((L,), jnp.float32)

    def _zrow(i, carry):
        for q in range(D_FEAT // L):
            rows_v[2, i, pl.ds(q * L, L)] = zero
        return carry

    lax.fori_loop(0, CHUNK, _zrow, 0)
    for r in range(OUT_PER_W // CHUNK):
        pltpu.sync_copy(rows_v.at[2], acc_sh.at[pl.ds(s * OUT_PER_W + r * CHUNK, CHUNK)])
    plsc.subcore_barrier()

    base_local = c * HALF
    for j in range(NCHUNK):
        if j + 2 < NCHUNK:
            desc[j + 2] = _gather(j + 2)
        for i in range(CHUNK // L):
            d = dst_v[pl.ds(j * CHUNK + i * L, L)]
            dl = d - base_local
            ok = (dl >= 0) & (dl < HALF)
            dloc_v[pl.ds(i * L, L)] = jnp.where(ok, dl, TRASH)
        desc[j].wait()
        pltpu.sync_copy(rows_v.at[j % 3], acc_sh.at[dloc_v], add=True)

    plsc.subcore_barrier()

    out_base = c * HALF + s * OUT_PER_W
    pltpu.sync_copy(
        acc_sh.at[pl.ds(s * OUT_PER_W, OUT_PER_W)],
        out_hbm.at[pl.ds(out_base, OUT_PER_W)],
    )


def kernel(x, readout_src, readout_dst):
    return _readout_kernel(
        x, readout_src.astype(jnp.int32), readout_dst.astype(jnp.int32)
    )
